# baseline (speedup 1.0000x reference)
import functools

import jax
import jax.numpy as jnp
from jax import lax
from jax.experimental import pallas as pl
from jax.experimental.pallas import tpu as pltpu
from jax.experimental.pallas import tpu_sc as plsc

_N = 10000
_DEG = 32
_D = 128
_NUM = 2
_R = _N * _NUM
_G = _R * _DEG

_SC_C = 80
_SC_K = 5
_SC_GRP = _SC_C * _SC_K
_NW = 32

_BLK = 200


def _sc_gather(table, idx):
  g_total = idx.shape[0]
  d = table.shape[1]
  per_w = g_total // _NW
  ngrp = per_w // _SC_GRP
  assert per_w % _SC_GRP == 0

  mesh = plsc.VectorSubcoreMesh(core_axis_name="c", subcore_axis_name="s")

  @functools.partial(
      pl.kernel,
      out_type=jax.ShapeDtypeStruct((g_total, d), jnp.float32),
      mesh=mesh,
      scratch_types=[
          pltpu.VMEM((_SC_GRP,), jnp.int32),
          pltpu.VMEM((_SC_GRP, d), jnp.float32),
          pltpu.SemaphoreType.DMA,
      ],
  )
  def gather_k(table_hbm, idx_hbm, out_hbm, idx_v, rows_v, gsem):
    wid = lax.axis_index("s") * 2 + lax.axis_index("c")
    base = wid * per_w

    def group(gi, carry):
      gbase = base + gi * _SC_GRP
      pltpu.sync_copy(idx_hbm.at[pl.ds(gbase, _SC_GRP)], idx_v)
      copies = []
      for j in range(_SC_K):
        copies.append(
            pltpu.async_copy(
                table_hbm.at[idx_v.at[pl.ds(j * _SC_C, _SC_C)]],
                rows_v.at[pl.ds(j * _SC_C, _SC_C)],
                gsem,
            ))
      for cp in copies:
        cp.wait()
      pltpu.sync_copy(rows_v, out_hbm.at[pl.ds(gbase, _SC_GRP)])
      return carry

    lax.fori_loop(0, ngrp, group, 0)

  return gather_k(table, idx)


def _tc_layer(h, m, ew, w_in, w_hh, bias, w_self, w_neigh, b_neigh):
  nblk = _R // _BLK

  def body(h_ref, m_ref, ew_ref, win_ref, whh_ref, b_ref, ws_ref, wn_ref,
           bn_ref, out_ref):
    h0 = h_ref[...]
    mm = (m_ref[...] * ew_ref[...][:, :, None]).astype(jnp.bfloat16)
    win = win_ref[...].astype(jnp.bfloat16)
    whh = whh_ref[...].astype(jnp.bfloat16)
    b = b_ref[...]
    xg = jnp.dot(mm.reshape(_BLK * _DEG, _D), win,
                 preferred_element_type=jnp.float32).reshape(_BLK, _DEG, 4 * _D)
    ht = jnp.zeros((_BLK, _D), jnp.float32)
    ct = jnp.zeros((_BLK, _D), jnp.float32)
    for t in range(_DEG):
      g = (xg[:, t, :]
           + jnp.dot(ht.astype(jnp.bfloat16), whh,
                     preferred_element_type=jnp.float32) + b)
      ig = jax.nn.sigmoid(g[:, :_D])
      fg = jax.nn.sigmoid(g[:, _D:2 * _D])
      gg = jnp.tanh(g[:, 2 * _D:3 * _D])
      og = jax.nn.sigmoid(g[:, 3 * _D:])
      ct = fg * ct + ig * gg
      ht = og * jnp.tanh(ct)
    out_ref[...] = (jnp.dot(h0, ws_ref[...], preferred_element_type=jnp.float32)
                    + jnp.dot(ht, wn_ref[...], preferred_element_type=jnp.float32)
                    + bn_ref[...])

  full = lambda i: (0, 0)
  return pl.pallas_call(
      body,
      grid=(nblk,),
      in_specs=[
          pl.BlockSpec((_BLK, _D), lambda i: (i, 0)),
          pl.BlockSpec((_BLK, _DEG, _D), lambda i: (i, 0, 0)),
          pl.BlockSpec((_BLK, _DEG), lambda i: (i, 0)),
          pl.BlockSpec((_D, 4 * _D), full),
          pl.BlockSpec((_D, 4 * _D), full),
          pl.BlockSpec((1, 4 * _D), full),
          pl.BlockSpec((_D, _D), full),
          pl.BlockSpec((_D, _D), full),
          pl.BlockSpec((1, _D), full),
      ],
      out_specs=pl.BlockSpec((_BLK, _D), lambda i: (i, 0)),
      out_shape=jax.ShapeDtypeStruct((_R, _D), jnp.float32),
  )(h, m, ew, w_in, w_hh, bias, w_self, w_neigh, b_neigh)


def kernel(x, nbr1, nbr2, nbr3, ew1, ew2, ew3,
           Wih1, Whh1, bih1, bhh1, Wself1, Wneigh1, bneigh1,
           Wih2, Whh2, bih2, bhh2, Wself2, Wneigh2, bneigh2,
           Wih3, Whh3, bih3, bhh3, Wself3, Wneigh3, bneigh3):
  h = x.reshape(_R, _D)
  coff = jnp.arange(_NUM, dtype=jnp.int32)[None, :, None]

  layers = []
  for nbr, ew, Wih, Whh, bih, bhh, Wself, Wneigh, bneigh in (
      (nbr1, ew1, Wih1, Whh1, bih1, bhh1, Wself1, Wneigh1, bneigh1),
      (nbr2, ew2, Wih2, Whh2, bih2, bhh2, Wself2, Wneigh2, bneigh2),
      (nbr3, ew3, Wih3, Whh3, bih3, bhh3, Wself3, Wneigh3, bneigh3)):
    idx = (nbr[:, None, :] * _NUM + coff).reshape(_G)
    ew_b = jnp.broadcast_to(ew[:, None, :], (_N, _NUM, _DEG)).reshape(_R, _DEG)
    layers.append((idx, ew_b, Wih.T, Whh.T, (bih + bhh)[None, :],
                   Wself.T, Wneigh.T, bneigh[None, :]))

  for idx, ew_b, w_in, w_hh, bias, w_self, w_neigh, b_neigh in layers:
    m = _sc_gather(h, idx).reshape(_R, _DEG, _D)
    h = _tc_layer(h, m, ew_b, w_in, w_hh, bias, w_self, w_neigh, b_neigh)

  return h.reshape(_N, _NUM, _D)


# speedup vs baseline: 3.1339x; 3.1339x over previous
import functools

import jax
import jax.numpy as jnp
from jax import lax
from jax.experimental import pallas as pl
from jax.experimental.pallas import tpu as pltpu
from jax.experimental.pallas import tpu_sc as plsc

_N = 10000
_DEG = 32
_D = 128
_NUM = 2
_R = _N * _NUM
_G = _R * _DEG

_SC_C = 80
_SC_K = 5
_SC_GRP = _SC_C * _SC_K
_NW = 32

_BLK = 200


def _sc_gather(table, idx):
  g_total = idx.shape[0]
  d = table.shape[1]
  per_w = g_total // _NW
  ngrp = per_w // _SC_GRP
  assert per_w % _SC_GRP == 0

  mesh = plsc.VectorSubcoreMesh(core_axis_name="c", subcore_axis_name="s")

  @functools.partial(
      pl.kernel,
      out_type=jax.ShapeDtypeStruct((g_total, d), jnp.float32),
      mesh=mesh,
      scratch_types=[
          pltpu.VMEM((_SC_GRP,), jnp.int32),
          pltpu.VMEM((_SC_GRP, d), jnp.float32),
          pltpu.SemaphoreType.DMA,
      ],
  )
  def gather_k(table_hbm, idx_hbm, out_hbm, idx_v, rows_v, gsem):
    wid = lax.axis_index("s") * 2 + lax.axis_index("c")
    base = wid * per_w

    def group(gi, carry):
      gbase = base + gi * _SC_GRP
      pltpu.sync_copy(idx_hbm.at[pl.ds(gbase, _SC_GRP)], idx_v)
      copies = []
      for j in range(_SC_K):
        copies.append(
            pltpu.async_copy(
                table_hbm.at[idx_v.at[pl.ds(j * _SC_C, _SC_C)]],
                rows_v.at[pl.ds(j * _SC_C, _SC_C)],
                gsem,
            ))
      for cp in copies:
        cp.wait()
      pltpu.sync_copy(rows_v, out_hbm.at[pl.ds(gbase, _SC_GRP)])
      return carry

    lax.fori_loop(0, ngrp, group, 0)

  return gather_k(table, idx)


def _sigm(v):
  return 0.5 + 0.5 * jnp.tanh(0.5 * v)


def _tc_layer(h, m, ew, w_gates, bias, w_sn, b_neigh):
  nblk = _R // _BLK

  def body(h_ref, m_ref, ew_ref, wg_ref, b_ref, wsn_ref, bn_ref, out_ref):
    h0 = h_ref[...]
    wg = wg_ref[...]
    b = b_ref[...]
    ht = jnp.zeros((_BLK, _D), jnp.float32)
    ct = jnp.zeros((_BLK, _D), jnp.float32)
    for t in range(_DEG):
      mt = m_ref[t] * ew_ref[t]
      g = jnp.dot(jnp.concatenate([mt, ht], axis=1), wg,
                  preferred_element_type=jnp.float32) + b
      ig = _sigm(g[:, :_D])
      fg = _sigm(g[:, _D:2 * _D])
      gg = jnp.tanh(g[:, 2 * _D:3 * _D])
      og = _sigm(g[:, 3 * _D:])
      ct = fg * ct + ig * gg
      ht = og * jnp.tanh(ct)
    out_ref[...] = (jnp.dot(jnp.concatenate([h0, ht], axis=1), wsn_ref[...],
                            preferred_element_type=jnp.float32) + bn_ref[...])

  full = lambda i: (0, 0)
  return pl.pallas_call(
      body,
      grid=(nblk,),
      in_specs=[
          pl.BlockSpec((_BLK, _D), lambda i: (i, 0)),
          pl.BlockSpec((_DEG, _BLK, _D), lambda i: (0, i, 0)),
          pl.BlockSpec((_DEG, _BLK, 1), lambda i: (0, i, 0)),
          pl.BlockSpec((2 * _D, 4 * _D), full),
          pl.BlockSpec((1, 4 * _D), full),
          pl.BlockSpec((2 * _D, _D), full),
          pl.BlockSpec((1, _D), full),
      ],
      out_specs=pl.BlockSpec((_BLK, _D), lambda i: (i, 0)),
      out_shape=jax.ShapeDtypeStruct((_R, _D), jnp.float32),
  )(h, m, ew, w_gates, bias, w_sn, b_neigh)


def kernel(x, nbr1, nbr2, nbr3, ew1, ew2, ew3,
           Wih1, Whh1, bih1, bhh1, Wself1, Wneigh1, bneigh1,
           Wih2, Whh2, bih2, bhh2, Wself2, Wneigh2, bneigh2,
           Wih3, Whh3, bih3, bhh3, Wself3, Wneigh3, bneigh3):
  h = x.reshape(_R, _D)
  coff = jnp.arange(_NUM, dtype=jnp.int32)[None, :, None]

  layers = []
  for nbr, ew, Wih, Whh, bih, bhh, Wself, Wneigh, bneigh in (
      (nbr1, ew1, Wih1, Whh1, bih1, bhh1, Wself1, Wneigh1, bneigh1),
      (nbr2, ew2, Wih2, Whh2, bih2, bhh2, Wself2, Wneigh2, bneigh2),
      (nbr3, ew3, Wih3, Whh3, bih3, bhh3, Wself3, Wneigh3, bneigh3)):
    idx = jnp.transpose(nbr[:, None, :] * _NUM + coff, (2, 0, 1)).reshape(_G)
    ew_b = jnp.transpose(
        jnp.broadcast_to(ew[:, None, :], (_N, _NUM, _DEG)),
        (2, 0, 1)).reshape(_DEG, _R, 1)
    w_gates = jnp.concatenate([Wih.T, Whh.T], axis=0)
    w_sn = jnp.concatenate([Wself.T, Wneigh.T], axis=0)
    layers.append((idx, ew_b, w_gates, (bih + bhh)[None, :],
                   w_sn, bneigh[None, :]))

  for idx, ew_b, w_gates, bias, w_sn, b_neigh in layers:
    m = _sc_gather(h, idx).reshape(_DEG, _R, _D)
    h = _tc_layer(h, m, ew_b, w_gates, bias, w_sn, b_neigh)

  return h.reshape(_N, _NUM, _D)


# speedup vs baseline: 7.1779x; 2.2904x over previous
import functools

import jax
import jax.numpy as jnp
from jax import lax
from jax.experimental import pallas as pl
from jax.experimental.pallas import tpu as pltpu
from jax.experimental.pallas import tpu_sc as plsc

_N = 10000
_DEG = 32
_D = 128
_NUM = 2
_G = _N * _DEG

_SC_C = 80
_SC_K = 5
_SC_GRP = _SC_C * _SC_K
_NW = 32

_BLK = 200


def _sc_gather(table, idx):
  g_total = idx.shape[0]
  d = table.shape[1]
  per_w = g_total // _NW
  ngrp = per_w // _SC_GRP
  assert per_w % _SC_GRP == 0

  mesh = plsc.VectorSubcoreMesh(core_axis_name="c", subcore_axis_name="s")

  @functools.partial(
      pl.kernel,
      out_type=jax.ShapeDtypeStruct((g_total, d), jnp.int32),
      mesh=mesh,
      scratch_types=[
          pltpu.VMEM((_SC_GRP,), jnp.int32),
          pltpu.VMEM((_SC_GRP, d), jnp.int32),
          pltpu.SemaphoreType.DMA,
      ],
  )
  def gather_k(table_hbm, idx_hbm, out_hbm, idx_v, rows_v, gsem):
    wid = lax.axis_index("s") * 2 + lax.axis_index("c")
    base = wid * per_w

    def group(gi, carry):
      gbase = base + gi * _SC_GRP
      pltpu.sync_copy(idx_hbm.at[pl.ds(gbase, _SC_GRP)], idx_v)
      copies = []
      for j in range(_SC_K):
        copies.append(
            pltpu.async_copy(
                table_hbm.at[idx_v.at[pl.ds(j * _SC_C, _SC_C)]],
                rows_v.at[pl.ds(j * _SC_C, _SC_C)],
                gsem,
            ))
      for cp in copies:
        cp.wait()
      pltpu.sync_copy(rows_v, out_hbm.at[pl.ds(gbase, _SC_GRP)])
      return carry

    lax.fori_loop(0, ngrp, group, 0)

  return gather_k(table, idx)


def _sigm(v):
  return 0.5 + 0.5 * jnp.tanh(0.5 * v)


def _unpack2(v32):
  lo = lax.bitcast_convert_type(lax.shift_left(v32, 16), jnp.float32)
  hi = lax.bitcast_convert_type(
      lax.bitwise_and(v32, jnp.int32(-65536)), jnp.float32)
  return lo, hi


def _pack2(f_lo, f_hi):
  def rne(f):
    u = lax.bitcast_convert_type(f, jnp.uint32)
    return u + jnp.uint32(0x7FFF) + (
        lax.shift_right_logical(u, jnp.uint32(16)) & jnp.uint32(1))
  lo = lax.shift_right_logical(rne(f_lo), jnp.uint32(16))
  hi = lax.bitwise_and(rne(f_hi), jnp.uint32(0xFFFF0000))
  return lax.bitcast_convert_type(lax.bitwise_or(lo, hi), jnp.int32)


def _tc_layer(hp, m, ew, w_gates, bias, w_sn, b_neigh, final):
  nblk = _N // _BLK
  b2 = 2 * _BLK

  def body(hp_ref, m_ref, ew_ref, wg_ref, b_ref, wsn_ref, bn_ref, *out_refs):
    h0lo, h0hi = _unpack2(hp_ref[...])
    h0 = jnp.concatenate([h0lo, h0hi], axis=0)
    wg = wg_ref[...]
    b = b_ref[...]
    ht = jnp.zeros((b2, _D), jnp.float32)
    ct = jnp.zeros((b2, _D), jnp.float32)
    for t in range(_DEG):
      et = ew_ref[t]
      mlo, mhi = _unpack2(m_ref[t])
      mt = jnp.concatenate([mlo * et, mhi * et], axis=0)
      g = jnp.dot(jnp.concatenate([mt, ht], axis=1), wg,
                  preferred_element_type=jnp.float32) + b
      ig = _sigm(g[:, :_D])
      fg = _sigm(g[:, _D:2 * _D])
      gg = jnp.tanh(g[:, 2 * _D:3 * _D])
      og = _sigm(g[:, 3 * _D:])
      ct = fg * ct + ig * gg
      ht = og * jnp.tanh(ct)
    out = (jnp.dot(jnp.concatenate([h0, ht], axis=1), wsn_ref[...],
                   preferred_element_type=jnp.float32) + bn_ref[...])
    if final:
      out_refs[0][...] = out[:_BLK]
      out_refs[1][...] = out[_BLK:]
    else:
      out_refs[0][...] = _pack2(out[:_BLK], out[_BLK:])

  full = lambda i: (0, 0)
  if final:
    out_specs = [pl.BlockSpec((_BLK, _D), lambda i: (i, 0)),
                 pl.BlockSpec((_BLK, _D), lambda i: (i, 0))]
    out_shape = [jax.ShapeDtypeStruct((_N, _D), jnp.float32),
                 jax.ShapeDtypeStruct((_N, _D), jnp.float32)]
  else:
    out_specs = pl.BlockSpec((_BLK, _D), lambda i: (i, 0))
    out_shape = jax.ShapeDtypeStruct((_N, _D), jnp.int32)
  return pl.pallas_call(
      body,
      grid=(nblk,),
      in_specs=[
          pl.BlockSpec((_BLK, _D), lambda i: (i, 0)),
          pl.BlockSpec((_DEG, _BLK, _D), lambda i: (0, i, 0)),
          pl.BlockSpec((_DEG, _BLK, 1), lambda i: (0, i, 0)),
          pl.BlockSpec((2 * _D, 4 * _D), full),
          pl.BlockSpec((1, 4 * _D), full),
          pl.BlockSpec((2 * _D, _D), full),
          pl.BlockSpec((1, _D), full),
      ],
      out_specs=out_specs,
      out_shape=out_shape,
  )(hp, m, ew, w_gates, bias, w_sn, b_neigh)


def kernel(x, nbr1, nbr2, nbr3, ew1, ew2, ew3,
           Wih1, Whh1, bih1, bhh1, Wself1, Wneigh1, bneigh1,
           Wih2, Whh2, bih2, bhh2, Wself2, Wneigh2, bneigh2,
           Wih3, Whh3, bih3, bhh3, Wself3, Wneigh3, bneigh3):
  xb = x.astype(jnp.bfloat16)
  hp = lax.bitcast_convert_type(jnp.transpose(xb, (0, 2, 1)), jnp.int32)

  layers = []
  for nbr, ew, Wih, Whh, bih, bhh, Wself, Wneigh, bneigh in (
      (nbr1, ew1, Wih1, Whh1, bih1, bhh1, Wself1, Wneigh1, bneigh1),
      (nbr2, ew2, Wih2, Whh2, bih2, bhh2, Wself2, Wneigh2, bneigh2),
      (nbr3, ew3, Wih3, Whh3, bih3, bhh3, Wself3, Wneigh3, bneigh3)):
    idx = jnp.transpose(nbr, (1, 0)).reshape(_G)
    ew_b = jnp.transpose(ew, (1, 0)).reshape(_DEG, _N, 1)
    w_gates = jnp.concatenate([Wih.T, Whh.T], axis=0)
    w_sn = jnp.concatenate([Wself.T, Wneigh.T], axis=0)
    layers.append((idx, ew_b, w_gates, (bih + bhh)[None, :],
                   w_sn, bneigh[None, :]))

  for li, (idx, ew_b, w_gates, bias, w_sn, b_neigh) in enumerate(layers):
    m = _sc_gather(hp, idx).reshape(_DEG, _N, _D)
    res = _tc_layer(hp, m, ew_b, w_gates, bias, w_sn, b_neigh,
                    final=(li == 2))
    hp = res

  c0, c1 = hp
  return jnp.stack([c0, c1], axis=1)


# speedup vs baseline: 7.2376x; 1.0083x over previous
import functools

import jax
import jax.numpy as jnp
from jax import lax
from jax.experimental import pallas as pl
from jax.experimental.pallas import tpu as pltpu
from jax.experimental.pallas import tpu_sc as plsc

_N = 10000
_DEG = 32
_D = 128
_NUM = 2
_G = _N * _DEG

_SC_C = 80
_SC_K = 5
_SC_GRP = _SC_C * _SC_K
_NW = 32

_BLK = 200


def _sc_gather(table, idx):
  g_total = idx.shape[0]
  d = table.shape[1]
  per_w = g_total // _NW
  ngrp = per_w // _SC_GRP
  assert per_w % _SC_GRP == 0

  mesh = plsc.VectorSubcoreMesh(core_axis_name="c", subcore_axis_name="s")

  @functools.partial(
      pl.kernel,
      out_type=jax.ShapeDtypeStruct((g_total, d), jnp.int32),
      mesh=mesh,
      scratch_types=[
          pltpu.VMEM((_SC_GRP,), jnp.int32),
          pltpu.VMEM((_SC_GRP, d), jnp.int32),
          pltpu.SemaphoreType.DMA,
      ],
  )
  def gather_k(table_hbm, idx_hbm, out_hbm, idx_v, rows_v, gsem):
    wid = lax.axis_index("s") * 2 + lax.axis_index("c")
    base = wid * per_w

    def group(gi, carry):
      gbase = base + gi * _SC_GRP
      pltpu.sync_copy(idx_hbm.at[pl.ds(gbase, _SC_GRP)], idx_v)
      copies = []
      for j in range(_SC_K):
        copies.append(
            pltpu.async_copy(
                table_hbm.at[idx_v.at[pl.ds(j * _SC_C, _SC_C)]],
                rows_v.at[pl.ds(j * _SC_C, _SC_C)],
                gsem,
            ))
      for cp in copies:
        cp.wait()
      pltpu.sync_copy(rows_v, out_hbm.at[pl.ds(gbase, _SC_GRP)])
      return carry

    lax.fori_loop(0, ngrp, group, 0)

  return gather_k(table, idx)


def _sigm(v):
  return 0.5 + 0.5 * jnp.tanh(0.5 * v)


def _unpack2(v32):
  lo = lax.bitcast_convert_type(lax.shift_left(v32, 16), jnp.float32)
  hi = lax.bitcast_convert_type(
      lax.bitwise_and(v32, jnp.int32(-65536)), jnp.float32)
  return lo, hi


def _pack2(f_lo, f_hi):
  def rne(f):
    u = lax.bitcast_convert_type(f, jnp.uint32)
    return u + jnp.uint32(0x7FFF) + (
        lax.shift_right_logical(u, jnp.uint32(16)) & jnp.uint32(1))
  lo = lax.shift_right_logical(rne(f_lo), jnp.uint32(16))
  hi = lax.bitwise_and(rne(f_hi), jnp.uint32(0xFFFF0000))
  return lax.bitcast_convert_type(lax.bitwise_or(lo, hi), jnp.int32)


def _tc_layer(hp, m, ew, w_gates, bias, w_sn, b_neigh, final):
  nblk = _N // _BLK
  b2 = 2 * _BLK

  def body(hp_ref, m_ref, ew_ref, wg_ref, b_ref, wsn_ref, bn_ref, *out_refs):
    h0lo, h0hi = _unpack2(hp_ref[...])
    h0 = jnp.concatenate([h0lo, h0hi], axis=0)
    wg = wg_ref[...]
    b = b_ref[...]
    ht = jnp.zeros((b2, _D), jnp.float32)
    ct = jnp.zeros((b2, _D), jnp.float32)
    for t in range(_DEG):
      et = ew_ref[t]
      mlo, mhi = _unpack2(m_ref[t])
      mt = jnp.concatenate([mlo * et, mhi * et], axis=0)
      g = jnp.dot(jnp.concatenate([mt, ht], axis=1).astype(jnp.bfloat16),
                  wg, preferred_element_type=jnp.float32) + b
      ig = 0.5 + 0.5 * jnp.tanh(g[:, :_D])
      fg = 0.5 + 0.5 * jnp.tanh(g[:, _D:2 * _D])
      gg = jnp.tanh(g[:, 2 * _D:3 * _D])
      og = 0.5 + 0.5 * jnp.tanh(g[:, 3 * _D:])
      ct = fg * ct + ig * gg
      ht = og * jnp.tanh(ct)
    out = (jnp.dot(jnp.concatenate([h0, ht], axis=1), wsn_ref[...],
                   preferred_element_type=jnp.float32) + bn_ref[...])
    if final:
      out_refs[0][...] = out[:_BLK]
      out_refs[1][...] = out[_BLK:]
    else:
      out_refs[0][...] = _pack2(out[:_BLK], out[_BLK:])

  full = lambda i: (0, 0)
  if final:
    out_specs = [pl.BlockSpec((_BLK, _D), lambda i: (i, 0)),
                 pl.BlockSpec((_BLK, _D), lambda i: (i, 0))]
    out_shape = [jax.ShapeDtypeStruct((_N, _D), jnp.float32),
                 jax.ShapeDtypeStruct((_N, _D), jnp.float32)]
  else:
    out_specs = pl.BlockSpec((_BLK, _D), lambda i: (i, 0))
    out_shape = jax.ShapeDtypeStruct((_N, _D), jnp.int32)
  return pl.pallas_call(
      body,
      grid=(nblk,),
      in_specs=[
          pl.BlockSpec((_BLK, _D), lambda i: (i, 0)),
          pl.BlockSpec((_DEG, _BLK, _D), lambda i: (0, i, 0)),
          pl.BlockSpec((_DEG, _BLK, 1), lambda i: (0, i, 0)),
          pl.BlockSpec((2 * _D, 4 * _D), full),
          pl.BlockSpec((1, 4 * _D), full),
          pl.BlockSpec((2 * _D, _D), full),
          pl.BlockSpec((1, _D), full),
      ],
      out_specs=out_specs,
      out_shape=out_shape,
  )(hp, m, ew, w_gates, bias, w_sn, b_neigh)


def kernel(x, nbr1, nbr2, nbr3, ew1, ew2, ew3,
           Wih1, Whh1, bih1, bhh1, Wself1, Wneigh1, bneigh1,
           Wih2, Whh2, bih2, bhh2, Wself2, Wneigh2, bneigh2,
           Wih3, Whh3, bih3, bhh3, Wself3, Wneigh3, bneigh3):
  xb = x.astype(jnp.bfloat16)
  hp = lax.bitcast_convert_type(jnp.transpose(xb, (0, 2, 1)), jnp.int32)

  layers = []
  for nbr, ew, Wih, Whh, bih, bhh, Wself, Wneigh, bneigh in (
      (nbr1, ew1, Wih1, Whh1, bih1, bhh1, Wself1, Wneigh1, bneigh1),
      (nbr2, ew2, Wih2, Whh2, bih2, bhh2, Wself2, Wneigh2, bneigh2),
      (nbr3, ew3, Wih3, Whh3, bih3, bhh3, Wself3, Wneigh3, bneigh3)):
    idx = jnp.transpose(nbr, (1, 0)).reshape(_G)
    ew_b = jnp.transpose(ew, (1, 0)).reshape(_DEG, _N, 1)
    gscale = jnp.concatenate(
        [jnp.full((2 * _D,), 0.5), jnp.ones((_D,)), jnp.full((_D,), 0.5)]
    ).astype(jnp.float32)[None, :]
    w_gates = (jnp.concatenate([Wih.T, Whh.T], axis=0)
               * gscale).astype(jnp.bfloat16)
    bias = ((bih + bhh)[None, :] * gscale)
    w_sn = jnp.concatenate([Wself.T, Wneigh.T], axis=0)
    layers.append((idx, ew_b, w_gates, bias, w_sn, bneigh[None, :]))

  for li, (idx, ew_b, w_gates, bias, w_sn, b_neigh) in enumerate(layers):
    m = _sc_gather(hp, idx).reshape(_DEG, _N, _D)
    res = _tc_layer(hp, m, ew_b, w_gates, bias, w_sn, b_neigh,
                    final=(li == 2))
    hp = res

  c0, c1 = hp
  return jnp.stack([c0, c1], axis=1)


# speedup vs baseline: 7.3184x; 1.0112x over previous
import functools

import jax
import jax.numpy as jnp
from jax import lax
from jax.experimental import pallas as pl
from jax.experimental.pallas import tpu as pltpu
from jax.experimental.pallas import tpu_sc as plsc

_N = 10000
_DEG = 32
_D = 128
_NUM = 2
_G = _N * _DEG

_SC_C = 80
_SC_K = 5
_SC_GRP = _SC_C * _SC_K
_NW = 32

_BLK = 200

_CHUNKS = ((0, 4000), (4000, 4000), (8000, 2000))


def _sc_gather(table, idx):
  g_total = idx.shape[0]
  d = table.shape[1]
  per_w = g_total // _NW
  ngrp = per_w // _SC_GRP
  assert per_w % _SC_GRP == 0

  mesh = plsc.VectorSubcoreMesh(core_axis_name="c", subcore_axis_name="s")

  @functools.partial(
      pl.kernel,
      out_type=jax.ShapeDtypeStruct((g_total, d), jnp.int32),
      mesh=mesh,
      scratch_types=[
          pltpu.VMEM((_SC_GRP,), jnp.int32),
          pltpu.VMEM((_SC_GRP, d), jnp.int32),
          pltpu.SemaphoreType.DMA,
      ],
  )
  def gather_k(table_hbm, idx_hbm, out_hbm, idx_v, rows_v, gsem):
    wid = lax.axis_index("s") * 2 + lax.axis_index("c")
    base = wid * per_w

    def group(gi, carry):
      gbase = base + gi * _SC_GRP
      pltpu.sync_copy(idx_hbm.at[pl.ds(gbase, _SC_GRP)], idx_v)
      copies = []
      for j in range(_SC_K):
        copies.append(
            pltpu.async_copy(
                table_hbm.at[idx_v.at[pl.ds(j * _SC_C, _SC_C)]],
                rows_v.at[pl.ds(j * _SC_C, _SC_C)],
                gsem,
            ))
      for cp in copies:
        cp.wait()
      pltpu.sync_copy(rows_v, out_hbm.at[pl.ds(gbase, _SC_GRP)])
      return carry

    lax.fori_loop(0, ngrp, group, 0)

  return gather_k(table, idx)


def _sigm(v):
  return 0.5 + 0.5 * jnp.tanh(0.5 * v)


def _unpack2(v32):
  lo = lax.bitcast_convert_type(lax.shift_left(v32, 16), jnp.float32)
  hi = lax.bitcast_convert_type(
      lax.bitwise_and(v32, jnp.int32(-65536)), jnp.float32)
  return lo, hi


def _pack2(f_lo, f_hi):
  def rne(f):
    u = lax.bitcast_convert_type(f, jnp.uint32)
    return u + jnp.uint32(0x7FFF) + (
        lax.shift_right_logical(u, jnp.uint32(16)) & jnp.uint32(1))
  lo = lax.shift_right_logical(rne(f_lo), jnp.uint32(16))
  hi = lax.bitwise_and(rne(f_hi), jnp.uint32(0xFFFF0000))
  return lax.bitcast_convert_type(lax.bitwise_or(lo, hi), jnp.int32)


def _tc_layer(hp, m, ew, w_gates, bias, w_sn, b_neigh, n0, nn, final):
  nblk = nn // _BLK
  blk0 = n0 // _BLK
  b2 = 2 * _BLK

  def body(hp_ref, m_ref, ew_ref, wg_ref, b_ref, wsn_ref, bn_ref, *out_refs):
    h0lo, h0hi = _unpack2(hp_ref[...])
    h0 = jnp.concatenate([h0lo, h0hi], axis=0)
    wg = wg_ref[...]
    b = b_ref[...]
    ht = jnp.zeros((b2, _D), jnp.float32)
    ct = jnp.zeros((b2, _D), jnp.float32)
    for t in range(_DEG):
      et = ew_ref[t]
      mlo, mhi = _unpack2(m_ref[t])
      mt = jnp.concatenate([mlo * et, mhi * et], axis=0)
      g = jnp.dot(jnp.concatenate([mt, ht], axis=1).astype(jnp.bfloat16),
                  wg, preferred_element_type=jnp.float32) + b
      ig = 0.5 + 0.5 * jnp.tanh(g[:, :_D])
      fg = 0.5 + 0.5 * jnp.tanh(g[:, _D:2 * _D])
      gg = jnp.tanh(g[:, 2 * _D:3 * _D])
      og = 0.5 + 0.5 * jnp.tanh(g[:, 3 * _D:])
      ct = fg * ct + ig * gg
      ht = og * jnp.tanh(ct)
    out = (jnp.dot(jnp.concatenate([h0, ht], axis=1), wsn_ref[...],
                   preferred_element_type=jnp.float32) + bn_ref[...])
    if final:
      out_refs[0][...] = out[:_BLK]
      out_refs[1][...] = out[_BLK:]
    else:
      out_refs[0][...] = _pack2(out[:_BLK], out[_BLK:])

  full = lambda i: (0, 0)
  if final:
    out_specs = [pl.BlockSpec((_BLK, _D), lambda i: (i, 0)),
                 pl.BlockSpec((_BLK, _D), lambda i: (i, 0))]
    out_shape = [jax.ShapeDtypeStruct((nn, _D), jnp.float32),
                 jax.ShapeDtypeStruct((nn, _D), jnp.float32)]
  else:
    out_specs = pl.BlockSpec((_BLK, _D), lambda i: (i, 0))
    out_shape = jax.ShapeDtypeStruct((nn, _D), jnp.int32)
  return pl.pallas_call(
      body,
      grid=(nblk,),
      in_specs=[
          pl.BlockSpec((_BLK, _D), lambda i: (blk0 + i, 0)),
          pl.BlockSpec((_DEG, _BLK, _D), lambda i: (0, i, 0)),
          pl.BlockSpec((_DEG, _BLK, 1), lambda i: (0, i, 0)),
          pl.BlockSpec((2 * _D, 4 * _D), full),
          pl.BlockSpec((1, 4 * _D), full),
          pl.BlockSpec((2 * _D, _D), full),
          pl.BlockSpec((1, _D), full),
      ],
      out_specs=out_specs,
      out_shape=out_shape,
  )(hp, m, ew, w_gates, bias, w_sn, b_neigh)


def kernel(x, nbr1, nbr2, nbr3, ew1, ew2, ew3,
           Wih1, Whh1, bih1, bhh1, Wself1, Wneigh1, bneigh1,
           Wih2, Whh2, bih2, bhh2, Wself2, Wneigh2, bneigh2,
           Wih3, Whh3, bih3, bhh3, Wself3, Wneigh3, bneigh3):
  xb = x.astype(jnp.bfloat16)
  hp = lax.bitcast_convert_type(jnp.transpose(xb, (0, 2, 1)), jnp.int32)

  layers = []
  for nbr, ew, Wih, Whh, bih, bhh, Wself, Wneigh, bneigh in (
      (nbr1, ew1, Wih1, Whh1, bih1, bhh1, Wself1, Wneigh1, bneigh1),
      (nbr2, ew2, Wih2, Whh2, bih2, bhh2, Wself2, Wneigh2, bneigh2),
      (nbr3, ew3, Wih3, Whh3, bih3, bhh3, Wself3, Wneigh3, bneigh3)):
    idx = [jnp.transpose(nbr[n0:n0 + nn], (1, 0)).reshape(_DEG * nn)
           for n0, nn in _CHUNKS]
    ew_b = [jnp.transpose(ew[n0:n0 + nn], (1, 0)).reshape(_DEG, nn, 1)
            for n0, nn in _CHUNKS]
    gscale = jnp.concatenate(
        [jnp.full((2 * _D,), 0.5), jnp.ones((_D,)), jnp.full((_D,), 0.5)]
    ).astype(jnp.float32)[None, :]
    w_gates = (jnp.concatenate([Wih.T, Whh.T], axis=0)
               * gscale).astype(jnp.bfloat16)
    bias = ((bih + bhh)[None, :] * gscale)
    w_sn = jnp.concatenate([Wself.T, Wneigh.T], axis=0)
    layers.append((idx, ew_b, w_gates, bias, w_sn, bneigh[None, :]))

  for li, (idx, ew_b, w_gates, bias, w_sn, b_neigh) in enumerate(layers):
    final = li == 2
    outs = []
    for ci, (n0, nn) in enumerate(_CHUNKS):
      m = _sc_gather(hp, idx[ci]).reshape(_DEG, nn, _D)
      outs.append(_tc_layer(hp, m, ew_b[ci], w_gates, bias, w_sn, b_neigh,
                            n0, nn, final))
    if final:
      c0 = jnp.concatenate([o[0] for o in outs], axis=0)
      c1 = jnp.concatenate([o[1] for o in outs], axis=0)
    else:
      hp = jnp.concatenate(outs, axis=0)

  return jnp.stack([c0, c1], axis=1)
